# baseline (speedup 1.0000x reference)
import jax
import jax.numpy as jnp
from jax import lax
from jax.experimental import pallas as pl
from jax.experimental.pallas import tpu as pltpu
from jax.experimental.pallas import tpu_sc as plsc

B = 16384
EMB = 32
NW = 32
CHUNK = 128
NCHUNK = B // CHUNK
CPW = NCHUNK // NW
NSEG = 28
GBYTES = NSEG * CHUNK * EMB * 4


def _sc_gather_body(bat_t, pit_t, team_t,
                    bat_id, base1, base2, base3, away_sb, home_sb,
                    pit_id, away_pit, home_pit,
                    fld_team, away_team, home_team,
                    x_h, idxbuf, gbuf, semi, semg, semw):
    wid = lax.axis_index("s") * 2 + lax.axis_index("c")

    singles = [bat_id, base1, base2, base3]

    def chunk_body(c_local, carry):
        r0 = (wid * CPW + c_local) * CHUNK
        rows = pl.ds(r0, CHUNK)

        def idx_dst(s):
            return idxbuf.at[pl.ds(s * CHUNK, CHUNK)]

        for s in range(4):
            pltpu.make_async_copy(singles[s].at[rows], idx_dst(s), semi).start()
        for j in range(9):
            pltpu.make_async_copy(away_sb.at[j, rows], idx_dst(4 + j), semi).start()
            pltpu.make_async_copy(home_sb.at[j, rows], idx_dst(13 + j), semi).start()
        for s, arr in ((22, pit_id), (23, away_pit), (24, home_pit),
                       (25, fld_team), (26, away_team), (27, home_team)):
            pltpu.make_async_copy(arr.at[rows], idx_dst(s), semi).start()
        pltpu.make_async_copy(bat_id.at[pl.ds(0, NSEG * CHUNK)], idxbuf, semi).wait()

        def table(s):
            return bat_t if s < 22 else (pit_t if s < 25 else team_t)

        for s in range(NSEG):
            pltpu.make_async_copy(
                table(s).at[idx_dst(s)],
                gbuf.at[pl.ds(s * CHUNK, CHUNK)], semg).start()
        pltpu.make_async_copy(
            x_h.at[0, pl.ds(0, NSEG * CHUNK), pl.ds(0, EMB)], gbuf, semg).wait()

        for s in range(NSEG):
            pltpu.make_async_copy(
                gbuf.at[pl.ds(s * CHUNK, CHUNK)],
                x_h.at[s // 4, rows, pl.ds((s % 4) * EMB, EMB)], semw).start()
        pltpu.make_async_copy(
            x_h.at[0, pl.ds(0, NSEG * CHUNK), pl.ds(0, EMB)], gbuf, semw).wait()
        return carry

    lax.fori_loop(0, CPW, chunk_body, 0)


_sc_gather = pl.kernel(
    _sc_gather_body,
    out_type=jax.ShapeDtypeStruct((7, B, 128), jnp.bfloat16),
    mesh=plsc.VectorSubcoreMesh(
        core_axis_name="c", subcore_axis_name="s",
        num_cores=2, num_subcores=16),
    scratch_types=[
        pltpu.VMEM((NSEG * CHUNK,), jnp.int32),
        pltpu.VMEM((NSEG * CHUNK, EMB), jnp.bfloat16),
        pltpu.SemaphoreType.DMA,
        pltpu.SemaphoreType.DMA,
        pltpu.SemaphoreType.DMA,
    ],
    compiler_params=pltpu.CompilerParams(use_tc_tiling_on_sc=False),
)


def _mlp_body(x, sc, w1, w1s, b1, w2, b2, wh, bh, o0, o1, o2, o3):
    bf16 = jnp.bfloat16
    xb = jnp.concatenate([x[t] for t in range(7)], axis=1)
    h1 = jnp.dot(xb, w1[...], preferred_element_type=jnp.float32)
    h1 = h1 + jnp.dot(sc[...].T.astype(bf16), w1s[...],
                      preferred_element_type=jnp.float32)
    h1 = jnp.maximum(h1 + b1[...], 0.0).astype(bf16)
    h2 = jnp.maximum(
        jnp.dot(h1, w2[...], preferred_element_type=jnp.float32) + b2[...],
        0.0).astype(bf16)
    lg = jnp.dot(h2, wh[...], preferred_element_type=jnp.float32) + bh[...]
    lgt = lg.T
    for i, o in enumerate((o0, o1, o2, o3)):
        sl = lgt[i * 5:(i + 1) * 5, :]
        m = jnp.max(sl, axis=0, keepdims=True)
        e = jnp.exp(sl - m)
        o[...] = e / jnp.sum(e, axis=0, keepdims=True)


def _mlp_call(BB, x, scal, W1p, W1s, b1r, W2, b2r, Wh, bhm):
    nblk = B // BB
    full = lambda shape: pl.BlockSpec(shape, lambda i: tuple(0 for _ in shape))
    return pl.pallas_call(
        _mlp_body,
        grid=(nblk,),
        in_specs=[
            pl.BlockSpec((7, BB, 128), lambda i: (0, i, 0)),
            pl.BlockSpec((8, BB), lambda i: (0, i)),
            full((896, 512)),
            full((8, 512)),
            full((1, 512)),
            full((512, 256)),
            full((1, 256)),
            full((256, 20)),
            full((1, 20)),
        ],
        out_specs=[pl.BlockSpec((5, BB), lambda i: (0, i))] * 4,
        out_shape=[jax.ShapeDtypeStruct((5, B), jnp.float32)] * 4,
    )(x, scal, W1p, W1s, b1r, W2, b2r, Wh, bhm)


def kernel(outs_ct, bat_id, pit_id, fld_team_id, base1_run_id, base2_run_id,
           base3_run_id, away_score_ct, home_score_ct, inn_ct, bat_home_id,
           away_bat_lineup, home_bat_lineup, away_start_bat_ids,
           home_start_bat_ids, away_pit_id, home_pit_id, away_team_id,
           home_team_id, bat_table, pit_table, team_table, W1, b1, W2, b2,
           Wbd, bbd, Wr1, br1, Wr2, br2, Wr3, br3):
    i32 = jnp.int32
    bf16_ = jnp.bfloat16
    x = _sc_gather(bat_table.astype(bf16_), pit_table.astype(bf16_),
                   team_table.astype(bf16_),
                   bat_id.astype(i32), base1_run_id.astype(i32),
                   base2_run_id.astype(i32), base3_run_id.astype(i32),
                   away_start_bat_ids.astype(i32).T, home_start_bat_ids.astype(i32).T,
                   pit_id.astype(i32), away_pit_id.astype(i32),
                   home_pit_id.astype(i32),
                   fld_team_id.astype(i32), away_team_id.astype(i32),
                   home_team_id.astype(i32))

    scal = jnp.concatenate(
        [outs_ct.T, away_score_ct.T, home_score_ct.T, inn_ct.T, bat_home_id.T,
         away_bat_lineup.T, home_bat_lineup.T,
         jnp.zeros((1, B), jnp.float32)], axis=0)

    bf16 = jnp.bfloat16
    W1p = jnp.concatenate(
        [W1[1:33], W1[97:193], W1[199:775],
         W1[33:65], W1[775:839],
         W1[65:97], W1[839:903]],
        axis=0).astype(bf16)
    W1s = jnp.concatenate([W1[0:1], W1[193:199],
                           jnp.zeros((1, 512), jnp.float32)],
                          axis=0).astype(bf16)
    Wh = jnp.concatenate([Wbd, Wr1, Wr2, Wr3], axis=1).astype(bf16)
    mask = jnp.array([0.0] * 11 + [-999.0, 0.0, 0.0, 0.0]
                     + [0.0, -999.0, -999.0, 0.0, 0.0], jnp.float32)
    bhm = (jnp.concatenate([bbd, br1, br2, br3]) + mask).reshape(1, 20)

    o0, o1, o2, o3 = _mlp_call(
        512, x, scal, W1p, W1s,
        b1.reshape(1, 512), W2.astype(bf16), b2.reshape(1, 256), Wh, bhm)
    return (o0.T, o1.T, o2.T, o3.T)


# speedup vs baseline: 1.1325x; 1.1325x over previous
import jax
import jax.numpy as jnp
from jax import lax
from jax.experimental import pallas as pl
from jax.experimental.pallas import tpu as pltpu
from jax.experimental.pallas import tpu_sc as plsc

B = 16384
EMB = 32
NW = 32
SPW = B // NW
CHUNK = 32
CPW = SPW // CHUNK
NSEG = 28
GROWS = NSEG * CHUNK


def _sc_gather_body(bat_t, pit_t, team_t,
                    bat_id, base1, base2, base3, away_sb, home_sb,
                    pit_id, away_pit, home_pit,
                    fld_team, away_team, home_team,
                    x_h, idxbuf, gbuf, semi, semg, semw):
    wid = lax.axis_index("s") * 2 + lax.axis_index("c")
    wbase = wid * SPW
    wrows = pl.ds(wbase, SPW)

    def idx_seg(s):
        return idxbuf.at[pl.ds(s * SPW, SPW)]

    singles = [bat_id, base1, base2, base3]
    for s in range(4):
        pltpu.make_async_copy(singles[s].at[wrows], idx_seg(s), semi).start()
    for j in range(9):
        pltpu.make_async_copy(away_sb.at[j, wrows], idx_seg(4 + j), semi).start()
        pltpu.make_async_copy(home_sb.at[j, wrows], idx_seg(13 + j), semi).start()
    for s, arr in ((22, pit_id), (23, away_pit), (24, home_pit),
                   (25, fld_team), (26, away_team), (27, home_team)):
        pltpu.make_async_copy(arr.at[wrows], idx_seg(s), semi).start()
    pltpu.make_async_copy(bat_id.at[pl.ds(0, NSEG * SPW)], idxbuf, semi).wait()

    def table(s):
        return bat_t if s < 22 else (pit_t if s < 25 else team_t)

    def chunk_body(c, carry):
        r0 = wbase + c * CHUNK

        for s in range(NSEG):
            pltpu.make_async_copy(
                table(s).at[idxbuf.at[pl.ds(s * SPW + c * CHUNK, CHUNK)]],
                gbuf.at[pl.ds(s * CHUNK, CHUNK)], semg).start()
        pltpu.make_async_copy(
            x_h.at[0, pl.ds(0, GROWS), :], gbuf, semg).wait()

        for s in range(NSEG):
            pltpu.make_async_copy(
                gbuf.at[pl.ds(s * CHUNK, CHUNK), pl.ds(0, EMB)],
                x_h.at[s // 4, pl.ds(r0, CHUNK), pl.ds((s % 4) * EMB, EMB)],
                semw).start()
        pltpu.make_async_copy(
            x_h.at[0, pl.ds(0, GROWS), pl.ds(0, EMB)],
            gbuf.at[:, pl.ds(0, EMB)], semw).wait()
        return carry

    lax.fori_loop(0, CPW, chunk_body, 0)


_sc_gather = pl.kernel(
    _sc_gather_body,
    out_type=jax.ShapeDtypeStruct((7, B, 128), jnp.float32),
    mesh=plsc.VectorSubcoreMesh(
        core_axis_name="c", subcore_axis_name="s",
        num_cores=2, num_subcores=16),
    scratch_types=[
        pltpu.VMEM((NSEG * SPW,), jnp.int32),
        pltpu.VMEM((GROWS, 128), jnp.float32),
        pltpu.SemaphoreType.DMA,
        pltpu.SemaphoreType.DMA,
        pltpu.SemaphoreType.DMA,
    ],
    compiler_params=pltpu.CompilerParams(use_tc_tiling_on_sc=False),
)


def _mlp_body(x, sc, w1, w1s, b1, w2, b2, wh, bh, o0, o1, o2, o3):
    bf16 = jnp.bfloat16
    xb = jnp.concatenate([x[t] for t in range(7)], axis=1).astype(bf16)
    h1 = jnp.dot(xb, w1[...], preferred_element_type=jnp.float32)
    h1 = h1 + jnp.dot(sc[...].T.astype(bf16), w1s[...],
                      preferred_element_type=jnp.float32)
    h1 = jnp.maximum(h1 + b1[...], 0.0).astype(bf16)
    h2 = jnp.maximum(
        jnp.dot(h1, w2[...], preferred_element_type=jnp.float32) + b2[...],
        0.0).astype(bf16)
    lg = jnp.dot(h2, wh[...], preferred_element_type=jnp.float32) + bh[...]
    lgt = lg.T
    for i, o in enumerate((o0, o1, o2, o3)):
        sl = lgt[i * 5:(i + 1) * 5, :]
        m = jnp.max(sl, axis=0, keepdims=True)
        e = jnp.exp(sl - m)
        o[...] = e / jnp.sum(e, axis=0, keepdims=True)


def _mlp_call(BB, x, scal, W1p, W1s, b1r, W2, b2r, Wh, bhm):
    nblk = B // BB
    full = lambda shape: pl.BlockSpec(shape, lambda i: tuple(0 for _ in shape))
    return pl.pallas_call(
        _mlp_body,
        grid=(nblk,),
        in_specs=[
            pl.BlockSpec((7, BB, 128), lambda i: (0, i, 0)),
            pl.BlockSpec((8, BB), lambda i: (0, i)),
            full((896, 512)),
            full((8, 512)),
            full((1, 512)),
            full((512, 256)),
            full((1, 256)),
            full((256, 20)),
            full((1, 20)),
        ],
        out_specs=[pl.BlockSpec((5, BB), lambda i: (0, i))] * 4,
        out_shape=[jax.ShapeDtypeStruct((5, B), jnp.float32)] * 4,
    )(x, scal, W1p, W1s, b1r, W2, b2r, Wh, bhm)


def kernel(outs_ct, bat_id, pit_id, fld_team_id, base1_run_id, base2_run_id,
           base3_run_id, away_score_ct, home_score_ct, inn_ct, bat_home_id,
           away_bat_lineup, home_bat_lineup, away_start_bat_ids,
           home_start_bat_ids, away_pit_id, home_pit_id, away_team_id,
           home_team_id, bat_table, pit_table, team_table, W1, b1, W2, b2,
           Wbd, bbd, Wr1, br1, Wr2, br2, Wr3, br3):
    i32 = jnp.int32
    pad = lambda t: jnp.pad(t, ((0, 0), (0, 128 - EMB)))
    x = _sc_gather(pad(bat_table), pad(pit_table), pad(team_table),
                   bat_id.astype(i32), base1_run_id.astype(i32),
                   base2_run_id.astype(i32), base3_run_id.astype(i32),
                   away_start_bat_ids.astype(i32).T, home_start_bat_ids.astype(i32).T,
                   pit_id.astype(i32), away_pit_id.astype(i32),
                   home_pit_id.astype(i32),
                   fld_team_id.astype(i32), away_team_id.astype(i32),
                   home_team_id.astype(i32))

    scal = jnp.concatenate(
        [outs_ct.T, away_score_ct.T, home_score_ct.T, inn_ct.T, bat_home_id.T,
         away_bat_lineup.T, home_bat_lineup.T,
         jnp.zeros((1, B), jnp.float32)], axis=0)

    bf16 = jnp.bfloat16
    W1p = jnp.concatenate(
        [W1[1:33], W1[97:193], W1[199:775],
         W1[33:65], W1[775:839],
         W1[65:97], W1[839:903]],
        axis=0).astype(bf16)
    W1s = jnp.concatenate([W1[0:1], W1[193:199],
                           jnp.zeros((1, 512), jnp.float32)],
                          axis=0).astype(bf16)
    Wh = jnp.concatenate([Wbd, Wr1, Wr2, Wr3], axis=1).astype(bf16)
    mask = jnp.array([0.0] * 11 + [-999.0, 0.0, 0.0, 0.0]
                     + [0.0, -999.0, -999.0, 0.0, 0.0], jnp.float32)
    bhm = (jnp.concatenate([bbd, br1, br2, br3]) + mask).reshape(1, 20)

    o0, o1, o2, o3 = _mlp_call(
        512, x, scal, W1p, W1s,
        b1.reshape(1, 512), W2.astype(bf16), b2.reshape(1, 256), Wh, bhm)
    return (o0.T, o1.T, o2.T, o3.T)


# speedup vs baseline: 1.7262x; 1.5243x over previous
import jax
import jax.numpy as jnp
from jax import lax
from jax.experimental import pallas as pl
from jax.experimental.pallas import tpu as pltpu
from jax.experimental.pallas import tpu_sc as plsc

B = 16384
EMB = 32
NW = 32
CHUNK = 128
NCHUNK = B // CHUNK
CPW = NCHUNK // NW

_SC_MESH = plsc.VectorSubcoreMesh(
    core_axis_name="c", subcore_axis_name="s", num_cores=2, num_subcores=16)


def _gather_chunks(nseg, idx_srcs, tables, x_h, idxbuf, gbuf, semi, semg, semw):
    wid = lax.axis_index("s") * 2 + lax.axis_index("c")

    def chunk_body(c_local, carry):
        r0 = (wid * CPW + c_local) * CHUNK
        rows = pl.ds(r0, CHUNK)

        def idx_dst(s):
            return idxbuf.at[pl.ds(s * CHUNK, CHUNK)]

        for s, src in enumerate(idx_srcs):
            if isinstance(src, tuple):
                arr, j = src
                pltpu.make_async_copy(arr.at[j, rows], idx_dst(s), semi).start()
            else:
                pltpu.make_async_copy(src.at[rows], idx_dst(s), semi).start()
        dummy_idx_src = idx_srcs[0]
        pltpu.make_async_copy(
            dummy_idx_src.at[pl.ds(0, nseg * CHUNK)], idxbuf, semi).wait()

        for s in range(nseg):
            pltpu.make_async_copy(
                tables[s].at[idx_dst(s)],
                gbuf.at[pl.ds(s * CHUNK, CHUNK)], semg).start()
        pltpu.make_async_copy(
            x_h.at[0, pl.ds(0, nseg * CHUNK), pl.ds(0, EMB)], gbuf, semg).wait()

        for s in range(nseg):
            pltpu.make_async_copy(
                gbuf.at[pl.ds(s * CHUNK, CHUNK)],
                x_h.at[s // 4, rows, pl.ds((s % 4) * EMB, EMB)], semw).start()
        pltpu.make_async_copy(
            x_h.at[0, pl.ds(0, nseg * CHUNK), pl.ds(0, EMB)], gbuf, semw).wait()
        return carry

    lax.fori_loop(0, CPW, chunk_body, 0)


def _sc_bat_body(bat_t, bat_id, base1, base2, base3, away_sb, home_sb,
                 x_h, idxbuf, gbuf, semi, semg, semw):
    idx_srcs = ([bat_id, base1, base2, base3]
                + [(away_sb, j) for j in range(9)]
                + [(home_sb, j) for j in range(9)]
                + [bat_id, bat_id])
    _gather_chunks(24, idx_srcs, [bat_t] * 24, x_h, idxbuf, gbuf,
                   semi, semg, semw)


def _sc_pitteam_body(pit_t, team_t, fld_team, away_team, home_team,
                     pit_id, away_pit, home_pit,
                     x_h, idxbuf, gbuf, semi, semg, semw):
    idx_srcs = [fld_team, away_team, home_team, fld_team,
                pit_id, away_pit, home_pit, pit_id]
    tables = [team_t] * 4 + [pit_t] * 4
    _gather_chunks(8, idx_srcs, tables, x_h, idxbuf, gbuf, semi, semg, semw)


def _make_sc(body, nseg, nplanes):
    return pl.kernel(
        body,
        out_type=jax.ShapeDtypeStruct((nplanes, B, 128), jnp.float32),
        mesh=_SC_MESH,
        scratch_types=[
            pltpu.VMEM((nseg * CHUNK,), jnp.int32),
            pltpu.VMEM((nseg * CHUNK, EMB), jnp.float32),
            pltpu.SemaphoreType.DMA,
            pltpu.SemaphoreType.DMA,
            pltpu.SemaphoreType.DMA,
        ],
        compiler_params=pltpu.CompilerParams(use_tc_tiling_on_sc=False),
    )


_sc_bat = _make_sc(_sc_bat_body, 24, 6)
_sc_pitteam = _make_sc(_sc_pitteam_body, 8, 2)


def _mlp_body(x2, x1, sc, w1, w1s, b1, w2, b2, wh, bh, o0, o1, o2, o3):
    bf16 = jnp.bfloat16
    xb = jnp.concatenate(
        [x1[t] for t in range(2)] + [x2[t] for t in range(6)],
        axis=1).astype(bf16)
    h1 = jnp.dot(xb, w1[...], preferred_element_type=jnp.float32)
    h1 = h1 + jnp.dot(sc[...].T.astype(bf16), w1s[...],
                      preferred_element_type=jnp.float32)
    h1 = jnp.maximum(h1 + b1[...], 0.0).astype(bf16)
    h2 = jnp.maximum(
        jnp.dot(h1, w2[...], preferred_element_type=jnp.float32) + b2[...],
        0.0).astype(bf16)
    lg = jnp.dot(h2, wh[...], preferred_element_type=jnp.float32) + bh[...]
    lgt = lg.T
    for i, o in enumerate((o0, o1, o2, o3)):
        sl = lgt[i * 5:(i + 1) * 5, :]
        m = jnp.max(sl, axis=0, keepdims=True)
        e = jnp.exp(sl - m)
        o[...] = e / jnp.sum(e, axis=0, keepdims=True)


def _mlp_call(BB, x2, x1, scal, W1p, W1s, b1r, W2, b2r, Wh, bhm):
    nblk = B // BB
    full = lambda shape: pl.BlockSpec(shape, lambda i: tuple(0 for _ in shape))
    return pl.pallas_call(
        _mlp_body,
        grid=(nblk,),
        in_specs=[
            pl.BlockSpec((6, BB, 128), lambda i: (0, i, 0)),
            pl.BlockSpec((2, BB, 128), lambda i: (0, i, 0)),
            pl.BlockSpec((8, BB), lambda i: (0, i)),
            full((1024, 512)),
            full((8, 512)),
            full((1, 512)),
            full((512, 256)),
            full((1, 256)),
            full((256, 20)),
            full((1, 20)),
        ],
        out_specs=[pl.BlockSpec((5, BB), lambda i: (0, i))] * 4,
        out_shape=[jax.ShapeDtypeStruct((5, B), jnp.float32)] * 4,
    )(x2, x1, scal, W1p, W1s, b1r, W2, b2r, Wh, bhm)


def kernel(outs_ct, bat_id, pit_id, fld_team_id, base1_run_id, base2_run_id,
           base3_run_id, away_score_ct, home_score_ct, inn_ct, bat_home_id,
           away_bat_lineup, home_bat_lineup, away_start_bat_ids,
           home_start_bat_ids, away_pit_id, home_pit_id, away_team_id,
           home_team_id, bat_table, pit_table, team_table, W1, b1, W2, b2,
           Wbd, bbd, Wr1, br1, Wr2, br2, Wr3, br3):
    i32 = jnp.int32
    x2 = _sc_bat(bat_table,
                 bat_id.astype(i32), base1_run_id.astype(i32),
                 base2_run_id.astype(i32), base3_run_id.astype(i32),
                 away_start_bat_ids.astype(i32).T,
                 home_start_bat_ids.astype(i32).T)
    x1 = _sc_pitteam(pit_table, team_table,
                     fld_team_id.astype(i32), away_team_id.astype(i32),
                     home_team_id.astype(i32),
                     pit_id.astype(i32), away_pit_id.astype(i32),
                     home_pit_id.astype(i32))

    scal = jnp.concatenate(
        [outs_ct.T, away_score_ct.T, home_score_ct.T, inn_ct.T, bat_home_id.T,
         away_bat_lineup.T, home_bat_lineup.T,
         jnp.zeros((1, B), jnp.float32)], axis=0)

    bf16 = jnp.bfloat16
    Z32 = jnp.zeros((32, 512), jnp.float32)
    W1p = jnp.concatenate(
        [W1[65:97], W1[839:903], Z32,
         W1[33:65], W1[775:839], Z32,
         W1[1:33], W1[97:193], W1[199:775],
         Z32, Z32],
        axis=0).astype(bf16)
    W1s = jnp.concatenate([W1[0:1], W1[193:199],
                           jnp.zeros((1, 512), jnp.float32)],
                          axis=0).astype(bf16)
    Wh = jnp.concatenate([Wbd, Wr1, Wr2, Wr3], axis=1).astype(bf16)
    mask = jnp.array([0.0] * 11 + [-999.0, 0.0, 0.0, 0.0]
                     + [0.0, -999.0, -999.0, 0.0, 0.0], jnp.float32)
    bhm = (jnp.concatenate([bbd, br1, br2, br3]) + mask).reshape(1, 20)

    o0, o1, o2, o3 = _mlp_call(
        512, x2, x1, scal, W1p, W1s,
        b1.reshape(1, 512), W2.astype(bf16), b2.reshape(1, 256), Wh, bhm)
    return (o0.T, o1.T, o2.T, o3.T)


# speedup vs baseline: 1.7492x; 1.0134x over previous
import jax
import jax.numpy as jnp
from jax import lax
from jax.experimental import pallas as pl
from jax.experimental.pallas import tpu as pltpu
from jax.experimental.pallas import tpu_sc as plsc

B = 16384
EMB = 32
NW = 32
CHUNK = 64
NCHUNK = B // CHUNK
CPW = NCHUNK // NW
NSEG = 28
GROWS = NSEG * CHUNK


def _sc_gather_body(bat_t, pit_t, team_t,
                    bat_id, base1, base2, base3, away_sb, home_sb,
                    pit_id, away_pit, home_pit,
                    fld_team, away_team, home_team,
                    x_h, idxA, idxB, gbufA, gbufB,
                    semi, semgA, semgB, semwA, semwB):
    wid = lax.axis_index("s") * 2 + lax.axis_index("c")

    singles = [bat_id, base1, base2, base3]

    def stage_idx(c, idxbuf):
        rows = pl.ds((wid * CPW + c) * CHUNK, CHUNK)

        def idx_dst(s):
            return idxbuf.at[pl.ds(s * CHUNK, CHUNK)]

        for s in range(4):
            pltpu.make_async_copy(singles[s].at[rows], idx_dst(s), semi).start()
        for j in range(9):
            pltpu.make_async_copy(away_sb.at[j, rows], idx_dst(4 + j), semi).start()
            pltpu.make_async_copy(home_sb.at[j, rows], idx_dst(13 + j), semi).start()
        for s, arr in ((22, pit_id), (23, away_pit), (24, home_pit),
                       (25, fld_team), (26, away_team), (27, home_team)):
            pltpu.make_async_copy(arr.at[rows], idx_dst(s), semi).start()
        pltpu.make_async_copy(bat_id.at[pl.ds(0, GROWS)], idxbuf, semi).wait()

    def table(s):
        return bat_t if s < 22 else (pit_t if s < 25 else team_t)

    def fire_gathers(idxbuf, gbuf, semg):
        for s in range(NSEG):
            pltpu.make_async_copy(
                table(s).at[idxbuf.at[pl.ds(s * CHUNK, CHUNK)]],
                gbuf.at[pl.ds(s * CHUNK, CHUNK)], semg).start()

    def drain_by_gbuf(gbuf, sem):
        pltpu.make_async_copy(
            x_h.at[0, pl.ds(0, GROWS), pl.ds(0, EMB)], gbuf, sem).wait()

    def fire_writebacks(c, gbuf, semw):
        rows = pl.ds((wid * CPW + c) * CHUNK, CHUNK)
        for s in range(NSEG):
            pltpu.make_async_copy(
                gbuf.at[pl.ds(s * CHUNK, CHUNK)],
                x_h.at[s // 4, rows, pl.ds((s % 4) * EMB, EMB)], semw).start()

    def pair_body(i, carry):
        c0 = 2 * i
        c1 = 2 * i + 1
        stage_idx(c0, idxA)

        @pl.when(i > 0)
        def _():
            drain_by_gbuf(gbufA, semwA)

        fire_gathers(idxA, gbufA, semgA)
        drain_by_gbuf(gbufA, semgA)
        fire_writebacks(c0, gbufA, semwA)

        stage_idx(c1, idxB)

        @pl.when(i > 0)
        def _():
            drain_by_gbuf(gbufB, semwB)

        fire_gathers(idxB, gbufB, semgB)
        drain_by_gbuf(gbufB, semgB)
        fire_writebacks(c1, gbufB, semwB)
        return carry

    lax.fori_loop(0, CPW // 2, pair_body, 0)
    drain_by_gbuf(gbufA, semwA)
    drain_by_gbuf(gbufB, semwB)


_sc_gather = pl.kernel(
    _sc_gather_body,
    out_type=jax.ShapeDtypeStruct((7, B, 128), jnp.float32),
    mesh=plsc.VectorSubcoreMesh(
        core_axis_name="c", subcore_axis_name="s",
        num_cores=2, num_subcores=16),
    scratch_types=[
        pltpu.VMEM((GROWS,), jnp.int32),
        pltpu.VMEM((GROWS,), jnp.int32),
        pltpu.VMEM((GROWS, EMB), jnp.float32),
        pltpu.VMEM((GROWS, EMB), jnp.float32),
        pltpu.SemaphoreType.DMA,
        pltpu.SemaphoreType.DMA,
        pltpu.SemaphoreType.DMA,
        pltpu.SemaphoreType.DMA,
        pltpu.SemaphoreType.DMA,
    ],
    compiler_params=pltpu.CompilerParams(use_tc_tiling_on_sc=False),
)


def _mlp_body(x, sc, w1, w1s, b1, w2, b2, wh, bh, o0, o1, o2, o3):
    bf16 = jnp.bfloat16
    xb = jnp.concatenate([x[t] for t in range(7)], axis=1).astype(bf16)
    h1 = jnp.dot(xb, w1[...], preferred_element_type=jnp.float32)
    h1 = h1 + jnp.dot(sc[...].T.astype(bf16), w1s[...],
                      preferred_element_type=jnp.float32)
    h1 = jnp.maximum(h1 + b1[...], 0.0).astype(bf16)
    h2 = jnp.maximum(
        jnp.dot(h1, w2[...], preferred_element_type=jnp.float32) + b2[...],
        0.0).astype(bf16)
    lg = jnp.dot(h2, wh[...], preferred_element_type=jnp.float32) + bh[...]
    lgt = lg.T
    for i, o in enumerate((o0, o1, o2, o3)):
        sl = lgt[i * 5:(i + 1) * 5, :]
        m = jnp.max(sl, axis=0, keepdims=True)
        e = jnp.exp(sl - m)
        o[...] = e / jnp.sum(e, axis=0, keepdims=True)


def _mlp_call(BB, x, scal, W1p, W1s, b1r, W2, b2r, Wh, bhm):
    nblk = B // BB
    full = lambda shape: pl.BlockSpec(shape, lambda i: tuple(0 for _ in shape))
    return pl.pallas_call(
        _mlp_body,
        grid=(nblk,),
        in_specs=[
            pl.BlockSpec((7, BB, 128), lambda i: (0, i, 0)),
            pl.BlockSpec((8, BB), lambda i: (0, i)),
            full((896, 512)),
            full((8, 512)),
            full((1, 512)),
            full((512, 256)),
            full((1, 256)),
            full((256, 20)),
            full((1, 20)),
        ],
        out_specs=[pl.BlockSpec((5, BB), lambda i: (0, i))] * 4,
        out_shape=[jax.ShapeDtypeStruct((5, B), jnp.float32)] * 4,
    )(x, scal, W1p, W1s, b1r, W2, b2r, Wh, bhm)


def kernel(outs_ct, bat_id, pit_id, fld_team_id, base1_run_id, base2_run_id,
           base3_run_id, away_score_ct, home_score_ct, inn_ct, bat_home_id,
           away_bat_lineup, home_bat_lineup, away_start_bat_ids,
           home_start_bat_ids, away_pit_id, home_pit_id, away_team_id,
           home_team_id, bat_table, pit_table, team_table, W1, b1, W2, b2,
           Wbd, bbd, Wr1, br1, Wr2, br2, Wr3, br3):
    i32 = jnp.int32
    x = _sc_gather(bat_table, pit_table, team_table,
                   bat_id.astype(i32), base1_run_id.astype(i32),
                   base2_run_id.astype(i32), base3_run_id.astype(i32),
                   away_start_bat_ids.astype(i32).T, home_start_bat_ids.astype(i32).T,
                   pit_id.astype(i32), away_pit_id.astype(i32),
                   home_pit_id.astype(i32),
                   fld_team_id.astype(i32), away_team_id.astype(i32),
                   home_team_id.astype(i32))

    scal = jnp.concatenate(
        [outs_ct.T, away_score_ct.T, home_score_ct.T, inn_ct.T, bat_home_id.T,
         away_bat_lineup.T, home_bat_lineup.T,
         jnp.zeros((1, B), jnp.float32)], axis=0)

    bf16 = jnp.bfloat16
    W1p = jnp.concatenate(
        [W1[1:33], W1[97:193], W1[199:775],
         W1[33:65], W1[775:839],
         W1[65:97], W1[839:903]],
        axis=0).astype(bf16)
    W1s = jnp.concatenate([W1[0:1], W1[193:199],
                           jnp.zeros((1, 512), jnp.float32)],
                          axis=0).astype(bf16)
    Wh = jnp.concatenate([Wbd, Wr1, Wr2, Wr3], axis=1).astype(bf16)
    mask = jnp.array([0.0] * 11 + [-999.0, 0.0, 0.0, 0.0]
                     + [0.0, -999.0, -999.0, 0.0, 0.0], jnp.float32)
    bhm = (jnp.concatenate([bbd, br1, br2, br3]) + mask).reshape(1, 20)

    o0, o1, o2, o3 = _mlp_call(
        512, x, scal, W1p, W1s,
        b1.reshape(1, 512), W2.astype(bf16), b2.reshape(1, 256), Wh, bhm)
    return (o0.T, o1.T, o2.T, o3.T)


# speedup vs baseline: 1.8497x; 1.0574x over previous
import jax
import jax.numpy as jnp
from jax import lax
from jax.experimental import pallas as pl
from jax.experimental.pallas import tpu as pltpu
from jax.experimental.pallas import tpu_sc as plsc

B = 16384
EMB = 32
NW = 32
CHUNK = 64
NCHUNK = B // CHUNK
CPW = NCHUNK // NW
NSEG = 28
GROWS = NSEG * CHUNK


def _sc_gather_body(bat_t, pit_t, team_t,
                    bat_id, base1, base2, base3, away_sb, home_sb,
                    pit_id, away_pit, home_pit,
                    fld_team, away_team, home_team,
                    x_h, idxA, idxB, gbufA, gbufB,
                    semi, semgA, semgB, semwA, semwB):
    wid = lax.axis_index("s") * 2 + lax.axis_index("c")

    singles = [bat_id, base1, base2, base3]

    def stage_idx(c, idxbuf):
        rows = pl.ds((wid * CPW + c) * CHUNK, CHUNK)

        def idx_dst(s):
            return idxbuf.at[pl.ds(s * CHUNK, CHUNK)]

        for s in range(4):
            pltpu.make_async_copy(singles[s].at[rows], idx_dst(s), semi).start()
        for j in range(9):
            pltpu.make_async_copy(away_sb.at[j, rows], idx_dst(4 + j), semi).start()
            pltpu.make_async_copy(home_sb.at[j, rows], idx_dst(13 + j), semi).start()
        for s, arr in ((22, pit_id), (23, away_pit), (24, home_pit),
                       (25, fld_team), (26, away_team), (27, home_team)):
            pltpu.make_async_copy(arr.at[rows], idx_dst(s), semi).start()
        pltpu.make_async_copy(bat_id.at[pl.ds(0, GROWS)], idxbuf, semi).wait()

    def table(s):
        return bat_t if s < 22 else (pit_t if s < 25 else team_t)

    def fire_gathers(idxbuf, gbuf, semg):
        for s in range(NSEG):
            pltpu.make_async_copy(
                table(s).at[idxbuf.at[pl.ds(s * CHUNK, CHUNK)]],
                gbuf.at[pl.ds(s * CHUNK, CHUNK)], semg).start()

    def drain_by_gbuf(gbuf, sem):
        pltpu.make_async_copy(
            x_h.at[0, pl.ds(0, GROWS), pl.ds(0, EMB)], gbuf, sem).wait()

    def fire_writebacks(c, gbuf, semw):
        rows = pl.ds((wid * CPW + c) * CHUNK, CHUNK)
        for s in range(NSEG):
            pltpu.make_async_copy(
                gbuf.at[pl.ds(s * CHUNK, CHUNK)],
                x_h.at[s // 4, rows, pl.ds((s % 4) * EMB, EMB)], semw).start()

    def pair_body(i, carry):
        c0 = 2 * i
        c1 = 2 * i + 1
        stage_idx(c0, idxA)

        @pl.when(i > 0)
        def _():
            drain_by_gbuf(gbufA, semwA)

        fire_gathers(idxA, gbufA, semgA)
        drain_by_gbuf(gbufA, semgA)
        fire_writebacks(c0, gbufA, semwA)

        stage_idx(c1, idxB)

        @pl.when(i > 0)
        def _():
            drain_by_gbuf(gbufB, semwB)

        fire_gathers(idxB, gbufB, semgB)
        drain_by_gbuf(gbufB, semgB)
        fire_writebacks(c1, gbufB, semwB)
        return carry

    lax.fori_loop(0, CPW // 2, pair_body, 0)
    drain_by_gbuf(gbufA, semwA)
    drain_by_gbuf(gbufB, semwB)


_sc_gather = pl.kernel(
    _sc_gather_body,
    out_type=jax.ShapeDtypeStruct((7, B, 128), jnp.float32),
    mesh=plsc.VectorSubcoreMesh(
        core_axis_name="c", subcore_axis_name="s",
        num_cores=2, num_subcores=16),
    scratch_types=[
        pltpu.VMEM((GROWS,), jnp.int32),
        pltpu.VMEM((GROWS,), jnp.int32),
        pltpu.VMEM((GROWS, EMB), jnp.float32),
        pltpu.VMEM((GROWS, EMB), jnp.float32),
        pltpu.SemaphoreType.DMA,
        pltpu.SemaphoreType.DMA,
        pltpu.SemaphoreType.DMA,
        pltpu.SemaphoreType.DMA,
        pltpu.SemaphoreType.DMA,
    ],
    compiler_params=pltpu.CompilerParams(use_tc_tiling_on_sc=False),
)


def _mlp_body(x, sc, w1, w1s, b1, w2, b2, wh, bh, o0, o1, o2, o3):
    bf16 = jnp.bfloat16
    xb = jnp.concatenate([x[t] for t in range(7)], axis=1).astype(bf16)
    h1 = jnp.dot(xb, w1[...], preferred_element_type=jnp.float32)
    h1 = h1 + jnp.dot(sc[...].T.astype(bf16), w1s[...],
                      preferred_element_type=jnp.float32)
    h1 = jnp.maximum(h1 + b1[...], 0.0).astype(bf16)
    h2 = jnp.maximum(
        jnp.dot(h1, w2[...], preferred_element_type=jnp.float32) + b2[...],
        0.0).astype(bf16)
    lg = jnp.dot(h2, wh[...], preferred_element_type=jnp.float32) + bh[...]
    lgt = lg.T
    for i, o in enumerate((o0, o1, o2, o3)):
        sl = lgt[i * 5:(i + 1) * 5, :]
        m = jnp.max(sl, axis=0, keepdims=True)
        e = jnp.exp(sl - m)
        o[...] = e / jnp.sum(e, axis=0, keepdims=True)


def _mlp_call(BB, x, scal, W1p, W1s, b1r, W2, b2r, Wh, bhm):
    nblk = B // BB
    full = lambda shape: pl.BlockSpec(shape, lambda i: tuple(0 for _ in shape))
    return pl.pallas_call(
        _mlp_body,
        grid=(nblk,),
        in_specs=[
            pl.BlockSpec((7, BB, 128), lambda i: (0, i, 0)),
            pl.BlockSpec((8, BB), lambda i: (0, i)),
            full((896, 512)),
            full((8, 512)),
            full((1, 512)),
            full((512, 256)),
            full((1, 256)),
            full((256, 20)),
            full((1, 20)),
        ],
        out_specs=[pl.BlockSpec((5, BB), lambda i: (0, i))] * 4,
        out_shape=[jax.ShapeDtypeStruct((5, B), jnp.float32)] * 4,
    )(x, scal, W1p, W1s, b1r, W2, b2r, Wh, bhm)


def kernel(outs_ct, bat_id, pit_id, fld_team_id, base1_run_id, base2_run_id,
           base3_run_id, away_score_ct, home_score_ct, inn_ct, bat_home_id,
           away_bat_lineup, home_bat_lineup, away_start_bat_ids,
           home_start_bat_ids, away_pit_id, home_pit_id, away_team_id,
           home_team_id, bat_table, pit_table, team_table, W1, b1, W2, b2,
           Wbd, bbd, Wr1, br1, Wr2, br2, Wr3, br3):
    i32 = jnp.int32
    x = _sc_gather(bat_table, pit_table, team_table,
                   bat_id.astype(i32), base1_run_id.astype(i32),
                   base2_run_id.astype(i32), base3_run_id.astype(i32),
                   away_start_bat_ids.astype(i32).T, home_start_bat_ids.astype(i32).T,
                   pit_id.astype(i32), away_pit_id.astype(i32),
                   home_pit_id.astype(i32),
                   fld_team_id.astype(i32), away_team_id.astype(i32),
                   home_team_id.astype(i32))

    scal = jnp.concatenate(
        [outs_ct.T, away_score_ct.T, home_score_ct.T, inn_ct.T, bat_home_id.T,
         away_bat_lineup.T, home_bat_lineup.T,
         jnp.zeros((1, B), jnp.float32)], axis=0)

    bf16 = jnp.bfloat16
    W1p = jnp.concatenate(
        [W1[1:33], W1[97:193], W1[199:775],
         W1[33:65], W1[775:839],
         W1[65:97], W1[839:903]],
        axis=0).astype(bf16)
    W1s = jnp.concatenate([W1[0:1], W1[193:199],
                           jnp.zeros((1, 512), jnp.float32)],
                          axis=0).astype(bf16)
    Wh = jnp.concatenate([Wbd, Wr1, Wr2, Wr3], axis=1).astype(bf16)
    mask = jnp.array([0.0] * 11 + [-999.0, 0.0, 0.0, 0.0]
                     + [0.0, -999.0, -999.0, 0.0, 0.0], jnp.float32)
    bhm = (jnp.concatenate([bbd, br1, br2, br3]) + mask).reshape(1, 20)

    o0, o1, o2, o3 = _mlp_call(
        2048, x, scal, W1p, W1s,
        b1.reshape(1, 512), W2.astype(bf16), b2.reshape(1, 256), Wh, bhm)
    return (o0.T, o1.T, o2.T, o3.T)
